# baseline (speedup 1.0000x reference)
import jax
import jax.numpy as jnp
from jax import lax
from jax.experimental import pallas as pl
from jax.experimental.pallas import tpu as pltpu

_H = 768
_E = 16
_TOPK = 2
_CAP = 256
_T = 2048
_TB = 256
_G = _T // _TB


def _router_body(x_ref, w1_ref, b1_ref, w2_ref, b2_ref, ew_ref,
                 disp_ref, comb_ref, probs_ref, aux_ref, acc_ref):
    i = pl.program_id(0)

    @pl.when(i == 0)
    def _init():
        acc_ref[...] = jnp.zeros_like(acc_ref)

    h = jnp.dot(x_ref[...], w1_ref[...], preferred_element_type=jnp.float32)
    h = jnp.maximum(h + b1_ref[...], 0.0)
    logits = jnp.dot(h, w2_ref[...], preferred_element_type=jnp.float32)
    logits = (logits + b2_ref[...]) * ew_ref[...]

    m = jnp.max(logits, axis=1, keepdims=True)
    ex = jnp.exp(logits - m)
    p = ex / jnp.sum(ex, axis=1, keepdims=True)
    probs_ref[...] = p

    iota_e = lax.broadcasted_iota(jnp.int32, (_TB, _E), 1)
    p0 = jnp.max(p, axis=1, keepdims=True)
    i0 = jnp.min(jnp.where(p == p0, iota_e, _E), axis=1, keepdims=True)
    oh0 = iota_e == i0
    pm = jnp.where(oh0, -1.0, p)
    p1 = jnp.max(pm, axis=1, keepdims=True)
    i1 = jnp.min(jnp.where(pm == p1, iota_e, _E), axis=1, keepdims=True)
    oh1 = iota_e == i1
    s = p0 + p1 + 1e-8
    w0 = p0 / s
    w1 = p1 / s

    oh0f = oh0.astype(jnp.float32)
    oh1f = oh1.astype(jnp.float32)
    s2 = oh0f + oh1f
    row = lax.broadcasted_iota(jnp.int32, (_TB, _TB), 0)
    col = lax.broadcasted_iota(jnp.int32, (_TB, _TB), 1)
    stril = (row > col).astype(jnp.float32)
    c_in = jnp.dot(stril, s2, preferred_element_type=jnp.float32)
    carry = acc_ref[0:1, 0:_E]
    c_tot = c_in + carry
    pos0 = jnp.sum(c_tot * oh0f, axis=1, keepdims=True).astype(jnp.int32)
    pos1 = jnp.sum(c_tot * oh1f, axis=1, keepdims=True).astype(jnp.int32)
    acc_ref[0:1, 0:_E] = carry + jnp.sum(s2, axis=0, keepdims=True)
    acc_ref[1:2, 0:_E] += jnp.sum(p, axis=0, keepdims=True)

    q0 = jnp.where(pos0 < _CAP, i0 * _CAP + pos0, -1)
    q1 = jnp.where(pos1 < _CAP, i1 * _CAP + pos1, -1)
    q0_3 = jnp.expand_dims(q0, 2)
    q1_3 = jnp.expand_dims(q1, 2)
    w0_3 = jnp.expand_dims(w0, 2)
    w1_3 = jnp.expand_dims(w1, 2)
    ee = lax.broadcasted_iota(jnp.int32, (_TB, _E, _CAP), 1)
    cc = lax.broadcasted_iota(jnp.int32, (_TB, _E, _CAP), 2)
    qq = ee * _CAP + cc
    m0 = qq == q0_3
    m1 = qq == q1_3
    disp_ref[...] = (m0 | m1).astype(jnp.float32)
    comb_ref[...] = jnp.where(m0, w0_3, jnp.where(m1, w1_3, 0.0))

    cnt = acc_ref[0:1, 0:_E]
    psum = acc_ref[1:2, 0:_E]
    aux_ref[...] = (0.1 * _E) * jnp.sum(
        (psum / _T) * (cnt / (_T * _TOPK)), axis=1, keepdims=True)


def kernel(hidden_states, W1, b1, W2, b2, expert_weights):
    Bv, Sv, Hv = hidden_states.shape
    x = hidden_states.reshape(Bv * Sv, Hv)

    disp, comb, probs, aux = pl.pallas_call(
        _router_body,
        grid=(_G,),
        in_specs=[
            pl.BlockSpec((_TB, _H), lambda i: (i, 0)),
            pl.BlockSpec((_H, _H), lambda i: (0, 0)),
            pl.BlockSpec((1, _H), lambda i: (0, 0)),
            pl.BlockSpec((_H, _E), lambda i: (0, 0)),
            pl.BlockSpec((1, _E), lambda i: (0, 0)),
            pl.BlockSpec((1, _E), lambda i: (0, 0)),
        ],
        out_specs=[
            pl.BlockSpec((_TB, _E, _CAP), lambda i: (i, 0, 0)),
            pl.BlockSpec((_TB, _E, _CAP), lambda i: (i, 0, 0)),
            pl.BlockSpec((_TB, _E), lambda i: (i, 0)),
            pl.BlockSpec((1, 1), lambda i: (0, 0)),
        ],
        out_shape=[
            jax.ShapeDtypeStruct((_T, _E, _CAP), jnp.float32),
            jax.ShapeDtypeStruct((_T, _E, _CAP), jnp.float32),
            jax.ShapeDtypeStruct((_T, _E), jnp.float32),
            jax.ShapeDtypeStruct((1, 1), jnp.float32),
        ],
        scratch_shapes=[pltpu.VMEM((8, 128), jnp.float32)],
    )(x, W1.T, b1.reshape(1, Hv), W2.T, b2.reshape(1, _E),
      expert_weights.reshape(1, _E))

    dispatch = disp.reshape(Bv, Sv, _E, _CAP)
    combine = comb.reshape(Bv, Sv, _E, _CAP)
    router_probs = probs.reshape(Bv, Sv, _E)
    return dispatch, combine, router_probs, aux.reshape(())
